# Initial kernel scaffold; baseline (speedup 1.0000x reference)
import jax
import jax.numpy as jnp
from jax.experimental import pallas as pl


def kernel(source_node_emb, target_node_emb, edge_label_index):
    raise NotImplementedError("write your pallas kernel here")


# speedup vs baseline: 1.1702x; 1.1702x over previous
import functools

import jax
import jax.numpy as jnp
from jax import lax
from jax.experimental import pallas as pl
from jax.experimental.pallas import tpu as pltpu
from jax.experimental.pallas import tpu_sc as plsc

N_NODES = 10000
D_FEAT = 128
N_EDGES = 320000

NUM_CORES = 2
NUM_SUBCORES = 16
NUM_WORKERS = NUM_CORES * NUM_SUBCORES
EDGES_PER_WORKER = N_EDGES // NUM_WORKERS
CHUNK = 80
NUM_CHUNKS = EDGES_PER_WORKER // CHUNK
GROUPS = CHUNK // 16


def _sc_kernel(src_emb, tgt_emb, src_idx, tgt_idx, out,
               idx_s_v, idx_t_v, rows_s, rows_t, out_v, sem):
    wid = lax.axis_index("s") * NUM_CORES + lax.axis_index("c")
    base = wid * EDGES_PER_WORKER

    pltpu.sync_copy(src_idx.at[pl.ds(base, EDGES_PER_WORKER)], idx_s_v)
    pltpu.sync_copy(tgt_idx.at[pl.ds(base, EDGES_PER_WORKER)], idx_t_v)

    lanes = lax.iota(jnp.int32, 16)

    def chunk_body(i, carry):
        off = i * CHUNK
        cp_s = pltpu.async_copy(
            src_emb.at[idx_s_v.at[pl.ds(off, CHUNK)]],
            rows_s, sem)
        cp_t = pltpu.async_copy(
            tgt_emb.at[idx_t_v.at[pl.ds(off, CHUNK)]],
            rows_t, sem)
        cp_s.wait()
        cp_t.wait()

        for g in range(GROUPS):
            row_ids = g * 16 + lanes

            def d_body(d, acc):
                dv = jnp.full((16,), d, dtype=jnp.int32)
                s = plsc.load_gather(rows_s, [row_ids, dv])
                t = plsc.load_gather(rows_t, [row_ids, dv])
                return acc + s * t

            acc = lax.fori_loop(0, D_FEAT, d_body,
                                jnp.zeros((16,), jnp.float32))
            out_v[pl.ds(g * 16, 16)] = acc

        pltpu.sync_copy(out_v, out.at[pl.ds(base + off, CHUNK)])
        return carry

    lax.fori_loop(0, NUM_CHUNKS, chunk_body, 0)


@jax.jit
def kernel(source_node_emb, target_node_emb, edge_label_index):
    mesh = plsc.VectorSubcoreMesh(core_axis_name="c", subcore_axis_name="s")
    k = functools.partial(
        pl.kernel,
        mesh=mesh,
        out_type=jax.ShapeDtypeStruct((N_EDGES,), jnp.float32),
        scratch_types=[
            pltpu.VMEM((EDGES_PER_WORKER,), jnp.int32),
            pltpu.VMEM((EDGES_PER_WORKER,), jnp.int32),
            pltpu.VMEM((CHUNK, D_FEAT), jnp.float32),
            pltpu.VMEM((CHUNK, D_FEAT), jnp.float32),
            pltpu.VMEM((CHUNK,), jnp.float32),
            pltpu.SemaphoreType.DMA,
        ],
        compiler_params=pltpu.CompilerParams(needs_layout_passes=False),
    )(_sc_kernel)
    return k(source_node_emb, target_node_emb,
             edge_label_index[0], edge_label_index[1])


# speedup vs baseline: 1.3386x; 1.1439x over previous
import functools

import jax
import jax.numpy as jnp
from jax import lax
from jax.experimental import pallas as pl
from jax.experimental.pallas import tpu as pltpu
from jax.experimental.pallas import tpu_sc as plsc

N_NODES = 10000
D_FEAT = 128
N_EDGES = 320000

NUM_CORES = 2
NUM_SUBCORES = 16
NUM_WORKERS = NUM_CORES * NUM_SUBCORES
EDGES_PER_WORKER = N_EDGES // NUM_WORKERS
CHUNK = 80
NUM_CHUNKS = EDGES_PER_WORKER // CHUNK
GROUPS = CHUNK // 16
UNROLL = 8


def _sc_kernel(src_emb, tgt_emb, src_idx, tgt_idx, out,
               idx_s_v, idx_t_v, rs0, rt0, rs1, rt1, out_v, sem0, sem1):
    wid = lax.axis_index("s") * NUM_CORES + lax.axis_index("c")
    base = wid * EDGES_PER_WORKER

    pltpu.sync_copy(src_idx.at[pl.ds(base, EDGES_PER_WORKER)], idx_s_v)
    pltpu.sync_copy(tgt_idx.at[pl.ds(base, EDGES_PER_WORKER)], idx_t_v)

    lanes = lax.iota(jnp.int32, 16)
    zf = jnp.zeros((16,), jnp.float32)
    zi = jnp.zeros((16,), jnp.int32)

    def fire(k, rs, rt, sem):
        off = k * CHUNK
        pltpu.async_copy(src_emb.at[idx_s_v.at[pl.ds(off, CHUNK)]], rs, sem)
        pltpu.async_copy(tgt_emb.at[idx_t_v.at[pl.ds(off, CHUNK)]], rt, sem)

    def wait(rs, rt, sem):
        pltpu.make_async_copy(src_emb.at[pl.ds(0, CHUNK)], rs, sem).wait()
        pltpu.make_async_copy(tgt_emb.at[pl.ds(0, CHUNK)], rt, sem).wait()

    def compute(k, rs, rt):
        off = k * CHUNK
        for g in range(GROUPS):
            row_ids = g * 16 + lanes

            def d_body(it, carry, rs=rs, rt=rt, row_ids=row_ids):
                a0, a1, a2, a3, dv = carry
                accs = [a0, a1, a2, a3]
                for j in range(UNROLL):
                    dvj = dv + j if j else dv
                    s = plsc.load_gather(rs, [row_ids, dvj])
                    t = plsc.load_gather(rt, [row_ids, dvj])
                    accs[j % 4] = accs[j % 4] + s * t
                return (*accs, dv + UNROLL)

            a0, a1, a2, a3, _ = lax.fori_loop(
                0, D_FEAT // UNROLL, d_body, (zf, zf, zf, zf, zi))
            out_v[pl.ds(g * 16, 16)] = (a0 + a1) + (a2 + a3)
        pltpu.sync_copy(out_v, out.at[pl.ds(base + off, CHUNK)])

    fire(0, rs0, rt0, sem0)

    @pl.loop(0, NUM_CHUNKS - 1, step=2)
    def _(k):
        fire(k + 1, rs1, rt1, sem1)
        wait(rs0, rt0, sem0)
        compute(k, rs0, rt0)
        fire(k + 2, rs0, rt0, sem0)
        wait(rs1, rt1, sem1)
        compute(k + 1, rs1, rt1)

    wait(rs0, rt0, sem0)
    compute(NUM_CHUNKS - 1, rs0, rt0)


@jax.jit
def kernel(source_node_emb, target_node_emb, edge_label_index):
    mesh = plsc.VectorSubcoreMesh(core_axis_name="c", subcore_axis_name="s")
    k = functools.partial(
        pl.kernel,
        mesh=mesh,
        out_type=jax.ShapeDtypeStruct((N_EDGES,), jnp.float32),
        scratch_types=[
            pltpu.VMEM((EDGES_PER_WORKER,), jnp.int32),
            pltpu.VMEM((EDGES_PER_WORKER,), jnp.int32),
            pltpu.VMEM((CHUNK, D_FEAT), jnp.float32),
            pltpu.VMEM((CHUNK, D_FEAT), jnp.float32),
            pltpu.VMEM((CHUNK, D_FEAT), jnp.float32),
            pltpu.VMEM((CHUNK, D_FEAT), jnp.float32),
            pltpu.VMEM((CHUNK,), jnp.float32),
            pltpu.SemaphoreType.DMA,
            pltpu.SemaphoreType.DMA,
        ],
        compiler_params=pltpu.CompilerParams(needs_layout_passes=False),
    )(_sc_kernel)
    return k(source_node_emb, target_node_emb,
             edge_label_index[0], edge_label_index[1])


# speedup vs baseline: 8.9607x; 6.6940x over previous
import functools

import jax
import jax.numpy as jnp
from jax import lax
from jax.experimental import pallas as pl
from jax.experimental.pallas import tpu as pltpu
from jax.experimental.pallas import tpu_sc as plsc

N_NODES = 10000
D_FEAT = 128
N_EDGES = 320000

NUM_CORES = 2
NUM_SUBCORES = 16
NUM_WORKERS = NUM_CORES * NUM_SUBCORES
EDGES_PER_WORKER = N_EDGES // NUM_WORKERS
CHUNK = 80
NUM_CHUNKS = EDGES_PER_WORKER // CHUNK
GROUPS = CHUNK // 16
UNROLL = 8


def _sc_kernel(src_emb, tgt_emb, src_idx, tgt_idx, out,
               idx_s_v, idx_t_v, rs0, rt0, rs1, rt1, out_v, sem0, sem1):
    wid = lax.axis_index("s") * NUM_CORES + lax.axis_index("c")
    base = wid * EDGES_PER_WORKER

    pltpu.sync_copy(src_idx.at[pl.ds(base, EDGES_PER_WORKER)], idx_s_v)
    pltpu.sync_copy(tgt_idx.at[pl.ds(base, EDGES_PER_WORKER)], idx_t_v)

    lanes = lax.iota(jnp.int32, 16)
    zf = jnp.zeros((16,), jnp.float32)
    zi = jnp.zeros((16,), jnp.int32)

    def fire(k, rs, rt, sem):
        off = k * CHUNK
        pltpu.async_copy(src_emb.at[idx_s_v.at[pl.ds(off, CHUNK)]], rs, sem)
        pltpu.async_copy(tgt_emb.at[idx_t_v.at[pl.ds(off, CHUNK)]], rt, sem)

    def wait(rs, rt, sem):
        pltpu.make_async_copy(src_emb.at[pl.ds(0, CHUNK)], rs, sem).wait()
        pltpu.make_async_copy(tgt_emb.at[pl.ds(0, CHUNK)], rt, sem).wait()

    def compute(k, rs, rt):
        off = k * CHUNK
        for g in range(GROUPS):
            row_ids = g * 16 + lanes

            def d_body(it, carry, rs=rs, rt=rt, row_ids=row_ids):
                a0, a1, a2, a3, dvl = carry
                accs = [a0, a1, a2, a3]
                for j in range(UNROLL):
                    col = (dvl + j) & (D_FEAT - 1) if j else dvl & (D_FEAT - 1)
                    s = plsc.load_gather(rs, [row_ids, col])
                    t = plsc.load_gather(rt, [row_ids, col])
                    accs[j % 4] = accs[j % 4] + s * t
                return (*accs, dvl + UNROLL)

            a0, a1, a2, a3, _ = lax.fori_loop(
                0, D_FEAT // UNROLL, d_body, (zf, zf, zf, zf, lanes))
            out_v[pl.ds(g * 16, 16)] = (a0 + a1) + (a2 + a3)
        pltpu.sync_copy(out_v, out.at[pl.ds(base + off, CHUNK)])

    fire(0, rs0, rt0, sem0)

    @pl.loop(0, NUM_CHUNKS - 1, step=2)
    def _(k):
        fire(k + 1, rs1, rt1, sem1)
        wait(rs0, rt0, sem0)
        compute(k, rs0, rt0)
        fire(k + 2, rs0, rt0, sem0)
        wait(rs1, rt1, sem1)
        compute(k + 1, rs1, rt1)

    wait(rs0, rt0, sem0)
    compute(NUM_CHUNKS - 1, rs0, rt0)


@jax.jit
def kernel(source_node_emb, target_node_emb, edge_label_index):
    mesh = plsc.VectorSubcoreMesh(core_axis_name="c", subcore_axis_name="s")
    k = functools.partial(
        pl.kernel,
        mesh=mesh,
        out_type=jax.ShapeDtypeStruct((N_EDGES,), jnp.float32),
        scratch_types=[
            pltpu.VMEM((EDGES_PER_WORKER,), jnp.int32),
            pltpu.VMEM((EDGES_PER_WORKER,), jnp.int32),
            pltpu.VMEM((CHUNK, D_FEAT), jnp.float32),
            pltpu.VMEM((CHUNK, D_FEAT), jnp.float32),
            pltpu.VMEM((CHUNK, D_FEAT), jnp.float32),
            pltpu.VMEM((CHUNK, D_FEAT), jnp.float32),
            pltpu.VMEM((CHUNK,), jnp.float32),
            pltpu.SemaphoreType.DMA,
            pltpu.SemaphoreType.DMA,
        ],
        compiler_params=pltpu.CompilerParams(needs_layout_passes=False),
    )(_sc_kernel)
    return k(source_node_emb, target_node_emb,
             edge_label_index[0], edge_label_index[1])
